# baseline (speedup 1.0000x reference)
import functools

import jax
import jax.numpy as jnp
from jax import lax
from jax.experimental import pallas as pl
from jax.experimental.pallas import tpu as pltpu
from jax.experimental.pallas import tpu_sc as plsc

BATCH = 16384
DIM = 64
NUM_CORES = 2
NUM_SUBCORES = 16
NUM_WORKERS = NUM_CORES * NUM_SUBCORES
B_PER_W = BATCH // NUM_WORKERS
CHUNK = 128
NCHUNK = B_PER_W // CHUNK

_mesh = plsc.VectorSubcoreMesh(core_axis_name="c", subcore_axis_name="s")


@functools.partial(
    pl.kernel,
    mesh=_mesh,
    out_type=jax.ShapeDtypeStruct((BATCH, DIM), jnp.float32),
    scratch_types=[
        pltpu.VMEM((NCHUNK, CHUNK), jnp.int32),
        pltpu.VMEM((B_PER_W, DIM), jnp.float32),
        pltpu.SemaphoreType.DMA,
    ],
    compiler_params=pltpu.CompilerParams(use_tc_tiling_on_sc=False),
)
def _sc_gather(table_hbm, idx_hbm, out_hbm, idx_v, rows_v, sem):
    wid = lax.axis_index("s") * NUM_CORES + lax.axis_index("c")
    pltpu.sync_copy(idx_hbm.at[pl.ds(wid * NCHUNK, NCHUNK)], idx_v)
    copies = [
        pltpu.async_copy(
            table_hbm.at[idx_v.at[j]],
            rows_v.at[pl.ds(j * CHUNK, CHUNK)],
            sem,
        )
        for j in range(NCHUNK)
    ]
    for c in copies:
        c.wait()
    pltpu.sync_copy(rows_v, out_hbm.at[pl.ds(wid * B_PER_W, B_PER_W)])


def kernel(embedding_weight, batch):
    idx = batch.astype(jnp.int32).reshape(BATCH // CHUNK, CHUNK)
    return _sc_gather(embedding_weight, idx)


# speedup vs baseline: 2.4012x; 2.4012x over previous
import functools

import jax
import jax.numpy as jnp
from jax import lax
from jax.experimental import pallas as pl
from jax.experimental.pallas import tpu as pltpu
from jax.experimental.pallas import tpu_sc as plsc

BATCH = 16384
DIM = 64
TILE = 8
NTILE_TABLE = 1500000 // TILE
NUM_CORES = 2
NUM_SUBCORES = 16
NUM_WORKERS = NUM_CORES * NUM_SUBCORES
B_PER_W = BATCH // NUM_WORKERS
GRP = 16
NGRP = B_PER_W // GRP

_mesh = plsc.VectorSubcoreMesh(core_axis_name="c", subcore_axis_name="s")


@functools.partial(
    pl.kernel,
    mesh=_mesh,
    out_type=jax.ShapeDtypeStruct((BATCH // TILE, TILE, DIM), jnp.float32),
    scratch_types=[
        pltpu.VMEM((B_PER_W,), jnp.int32),
        pltpu.VMEM((GRP, TILE, DIM), jnp.float32),
        pltpu.VMEM((GRP, TILE, DIM), jnp.float32),
        pltpu.VMEM((B_PER_W // TILE, TILE, DIM), jnp.float32),
        pltpu.SemaphoreType.DMA,
    ],
)
def _sc_gather(table_hbm, idx_hbm, out_hbm, idx_v, gbuf0, gbuf1, oslab, sem):
    wid = lax.axis_index("s") * NUM_CORES + lax.axis_index("c")
    base = wid * B_PER_W
    pltpu.sync_copy(idx_hbm.at[pl.ds(base, B_PER_W)], idx_v)

    def fire(g, gbuf):
        ids = idx_v[pl.ds(g * GRP, GRP)]
        tvec = ids >> 3
        copies = [
            pltpu.async_copy(
                table_hbm.at[pl.ds(tvec[lane], 1)],
                gbuf.at[pl.ds(lane, 1)],
                sem,
            )
            for lane in range(GRP)
        ]
        return copies

    def select(g, gbuf, copies):
        ids = idx_v[pl.ds(g * GRP, GRP)]
        rvec = ids & 7
        for c in copies:
            c.wait()
        for lane in range(GRP):
            r = rvec[lane]
            for k in range(DIM // 16):
                oslab[2 * g + lane // TILE, lane % TILE,
                      pl.ds(k * 16, 16)] = gbuf[lane, r, pl.ds(k * 16, 16)]

    def body(i, carry):
        g0 = 2 * i
        g1 = 2 * i + 1
        c0 = fire(g0, gbuf0)
        c1 = fire(g1, gbuf1)
        select(g0, gbuf0, c0)
        select(g1, gbuf1, c1)
        return carry

    lax.fori_loop(0, NGRP // 2, body, 0)
    pltpu.sync_copy(
        oslab, out_hbm.at[pl.ds(wid * (B_PER_W // TILE), B_PER_W // TILE)])


def kernel(embedding_weight, batch):
    table3 = embedding_weight.reshape(NTILE_TABLE, TILE, DIM)
    idx = batch.astype(jnp.int32)
    out3 = _sc_gather(table3, idx)
    return out3.reshape(BATCH, DIM)
